# baseline (speedup 1.0000x reference)
import functools

import jax
import jax.numpy as jnp
from jax import lax
from jax.experimental import pallas as pl
from jax.experimental.pallas import tpu as pltpu
from jax.experimental.pallas import tpu_sc as plsc

N = 100000
E = 6400000
TD = 16

NWORKERS = 32
ZCH = 6272
NF = 16 * ZCH
K = 2048
TOTAL_CHUNKS = E // K
MAXT = -(-TOTAL_CHUNKS // NWORKERS)
MAXT_PAD = 99
XTAIL = N - 15 * ZCH

_mesh = plsc.VectorSubcoreMesh(core_axis_name="c", subcore_axis_name="s")


@functools.partial(
    pl.kernel,
    out_type=jax.ShapeDtypeStruct((3, NF), jnp.float32),
    mesh=_mesh,
    compiler_params=pltpu.CompilerParams(
        needs_layout_passes=False,
        use_tc_tiling_on_sc=False,
    ),
    scratch_types=[
        pltpu.VMEM((N,), jnp.float32),
        pltpu.VMEM((K,), jnp.int32),
        pltpu.VMEM((K,), jnp.int32),
        pltpu.VMEM((K,), jnp.int32),
        pltpu.VMEM((K,), jnp.int32),
        pltpu.VMEM((K,), jnp.int32),
        pltpu.VMEM((K,), jnp.int32),
        pltpu.VMEM((K,), jnp.float32),
        pltpu.VMEM((K,), jnp.float32),
        pltpu.VMEM((K,), jnp.float32),
        pltpu.VMEM((ZCH,), jnp.float32),
        pltpu.VMEM_SHARED((NF,), jnp.float32),
        pltpu.SemaphoreType.DMA,
        pltpu.SemaphoreType.DMA,
        pltpu.SemaphoreType.DMA,
        pltpu.SemaphoreType.DMA,
        pltpu.SemaphoreType.DMA,
        pltpu.SemaphoreType.DMA,
        pltpu.SemaphoreType.DMA,
    ],
)
def _sc_field(x_hbm, row_hbm, col_hbm, out_hbm, xv, rowv0, rowv1, rowv2,
              colv0, colv1, colv2, bond0, bond1, bond2, zero_v, field_sp,
              sem_x, sem_i0, sem_i1, sem_i2, sem_s0, sem_s1, sem_s2):
    c = lax.axis_index("c")
    s = lax.axis_index("s")
    wid = s * 2 + c

    row_bufs = (rowv0, rowv1, rowv2)
    col_bufs = (colv0, colv1, colv2)
    bond_bufs = (bond0, bond1, bond2)
    isems = (sem_i0, sem_i1, sem_i2)
    ssems = (sem_s0, sem_s1, sem_s2)

    cp_x = pltpu.async_copy(x_hbm, xv, sem_x)
    pltpu.async_copy(row_hbm.at[pl.ds(wid * K, K)], rowv0, sem_i0)
    pltpu.async_copy(col_hbm.at[pl.ds(wid * K, K)], colv0, sem_i0)

    zeros16 = jnp.zeros((16,), jnp.float32)

    def _zbody(i, carry):
        zero_v[pl.ds(i * 16, 16)] = zeros16
        return carry

    lax.fori_loop(0, ZCH // 16, _zbody, 0)
    pltpu.sync_copy(zero_v, field_sp.at[pl.ds(s * ZCH, ZCH)])
    cp_x.wait()
    plsc.subcore_barrier()

    def _phase(t, b):
        ch = wid + NWORKERS * t
        valid = ch < TOTAL_CHUNKS
        bn = (b + 1) % 3
        row_v = row_bufs[b]
        col_v = col_bufs[b]
        bond_v = bond_bufs[b]

        @pl.when(valid)
        def _():
            pltpu.make_async_copy(
                row_hbm.at[pl.ds(0, K)], row_v, isems[b]).wait()
            pltpu.make_async_copy(
                col_hbm.at[pl.ds(0, K)], col_v, isems[b]).wait()

            def _g(i, inner):
                for j in range(8):
                    o = i * 128 + j * 16
                    r = row_v[pl.ds(o, 16)]
                    cc = col_v[pl.ds(o, 16)]
                    xa = plsc.load_gather(xv, [r])
                    xb = plsc.load_gather(xv, [cc])
                    bond_v[pl.ds(o, 16)] = xa * xb
                return inner

            lax.fori_loop(0, K // 128, _g, 0)

        @pl.when((t > 1) & (ch - 2 * NWORKERS < TOTAL_CHUNKS))
        def _():
            pltpu.make_async_copy(
                x_hbm.at[pl.ds(0, K)], bond_bufs[bn], ssems[bn]).wait()

        @pl.when(ch + NWORKERS < TOTAL_CHUNKS)
        def _():
            e0 = (ch + NWORKERS) * K
            pltpu.async_copy(
                row_hbm.at[pl.ds(e0, K)], row_bufs[bn], isems[bn])
            pltpu.async_copy(
                col_hbm.at[pl.ds(e0, K)], col_bufs[bn], isems[bn])

        @pl.when(valid)
        def _():
            pltpu.async_copy(bond_v, field_sp.at[row_v], ssems[b], add=True)

    def _triple(tp, carry):
        _phase(tp * 3, 0)
        _phase(tp * 3 + 1, 1)
        _phase(tp * 3 + 2, 2)
        return carry

    lax.fori_loop(0, MAXT_PAD // 3, _triple, 0)

    @pl.when(wid + NWORKERS * (MAXT - 1) < TOTAL_CHUNKS)
    def _():
        pltpu.make_async_copy(
            x_hbm.at[pl.ds(0, K)], bond_bufs[(MAXT - 1) % 3],
            ssems[(MAXT - 1) % 3]).wait()

    @pl.when((c == 0) & (s < 15))
    def _():
        pltpu.sync_copy(xv.at[pl.ds(s * ZCH, ZCH)],
                        out_hbm.at[2, pl.ds(s * ZCH, ZCH)])

    @pl.when((c == 0) & (s == 15))
    def _():
        pltpu.sync_copy(xv.at[pl.ds(15 * ZCH, XTAIL)],
                        out_hbm.at[2, pl.ds(15 * ZCH, XTAIL)])

    plsc.subcore_barrier()
    pltpu.sync_copy(field_sp.at[pl.ds(s * ZCH, ZCH)],
                    out_hbm.at[c, pl.ds(s * ZCH, ZCH)])


BN = 1024
GRID = -(-N // BN)


def _mlp_body(p_ref, w1_ref, b1_ref, w2_ref, b2_ref, o_ref):
    p = p_ref[...]
    feats = jnp.concatenate(
        [p[2:3, :], p[0:1, :] + p[1:2, :]], axis=0)
    h = lax.dot_general(w1_ref[...], feats, (((0,), (0,)), ((), ())),
                        preferred_element_type=jnp.float32)
    h = jnp.maximum(h + b1_ref[...], 0.0)
    o = lax.dot_general(w2_ref[...], h, (((0,), (0,)), ((), ())),
                        preferred_element_type=jnp.float32)
    o = jnp.maximum(o + b2_ref[...], 0.0)
    o_ref[...] = o.T


_mlp = pl.pallas_call(
    _mlp_body,
    grid=(GRID,),
    in_specs=[
        pl.BlockSpec((3, BN), lambda i: (0, i)),
        pl.BlockSpec((2, TD), lambda i: (0, 0)),
        pl.BlockSpec((TD, 1), lambda i: (0, 0)),
        pl.BlockSpec((TD, TD), lambda i: (0, 0)),
        pl.BlockSpec((TD, 1), lambda i: (0, 0)),
    ],
    out_specs=pl.BlockSpec((BN, TD), lambda i: (i, 0)),
    out_shape=jax.ShapeDtypeStruct((N, TD), jnp.float32),
)


def kernel(x, edge_index, W1, b1, W2, b2):
    e = edge_index.astype(jnp.int32)
    part = _sc_field(x.reshape((N,)), e[0], e[1])
    return _mlp(part, W1, b1.reshape(TD, 1), W2, b2.reshape(TD, 1))


# speedup vs baseline: 1.1577x; 1.1577x over previous
import functools

import jax
import jax.numpy as jnp
from jax import lax
from jax.experimental import pallas as pl
from jax.experimental.pallas import tpu as pltpu
from jax.experimental.pallas import tpu_sc as plsc

N = 100000
E = 6400000
TD = 16

NWORKERS = 32
ZCH = 6272
NF = 16 * ZCH
K = 2048
TOTAL_CHUNKS = E // K
MAXT = -(-TOTAL_CHUNKS // NWORKERS)
MAXT_PAD = 99
XTAIL = N - 15 * ZCH

_mesh = plsc.VectorSubcoreMesh(core_axis_name="c", subcore_axis_name="s")


@functools.partial(
    pl.kernel,
    out_type=jax.ShapeDtypeStruct((3, NF), jnp.float32),
    mesh=_mesh,
    compiler_params=pltpu.CompilerParams(
        needs_layout_passes=False,
        use_tc_tiling_on_sc=False,
    ),
    scratch_types=[
        pltpu.VMEM((N,), jnp.float32),
        pltpu.VMEM((K,), jnp.int32),
        pltpu.VMEM((K,), jnp.int32),
        pltpu.VMEM((K,), jnp.int32),
        pltpu.VMEM((K,), jnp.int32),
        pltpu.VMEM((K,), jnp.int32),
        pltpu.VMEM((K,), jnp.int32),
        pltpu.VMEM((K,), jnp.float32),
        pltpu.VMEM((K,), jnp.float32),
        pltpu.VMEM((K,), jnp.float32),
        pltpu.VMEM((ZCH,), jnp.float32),
        pltpu.VMEM_SHARED((NF,), jnp.float32),
        pltpu.SemaphoreType.DMA,
        pltpu.SemaphoreType.DMA,
        pltpu.SemaphoreType.DMA,
        pltpu.SemaphoreType.DMA,
        pltpu.SemaphoreType.DMA,
        pltpu.SemaphoreType.DMA,
        pltpu.SemaphoreType.DMA,
    ],
)
def _sc_field(x_hbm, e_hbm, out_hbm, xv, rowv0, rowv1, rowv2,
              colv0, colv1, colv2, bond0, bond1, bond2, zero_v, field_sp,
              sem_x, sem_i0, sem_i1, sem_i2, sem_s0, sem_s1, sem_s2):
    c = lax.axis_index("c")
    s = lax.axis_index("s")
    wid = s * 2 + c

    row_bufs = (rowv0, rowv1, rowv2)
    col_bufs = (colv0, colv1, colv2)
    bond_bufs = (bond0, bond1, bond2)
    isems = (sem_i0, sem_i1, sem_i2)
    ssems = (sem_s0, sem_s1, sem_s2)

    cp_x = pltpu.async_copy(x_hbm, xv, sem_x)
    pltpu.async_copy(e_hbm.at[0, pl.ds(wid * K, K)], rowv0, sem_i0)
    pltpu.async_copy(e_hbm.at[1, pl.ds(wid * K, K)], colv0, sem_i0)

    zeros16 = jnp.zeros((16,), jnp.float32)

    def _zbody(i, carry):
        zero_v[pl.ds(i * 16, 16)] = zeros16
        return carry

    lax.fori_loop(0, ZCH // 16, _zbody, 0)
    pltpu.sync_copy(zero_v, field_sp.at[pl.ds(s * ZCH, ZCH)])
    cp_x.wait()
    plsc.subcore_barrier()

    def _phase(t, b):
        ch = wid + NWORKERS * t
        valid = ch < TOTAL_CHUNKS
        bn = (b + 1) % 3
        row_v = row_bufs[b]
        col_v = col_bufs[b]
        bond_v = bond_bufs[b]

        @pl.when(valid)
        def _():
            pltpu.make_async_copy(
                e_hbm.at[0, pl.ds(0, K)], row_v, isems[b]).wait()
            pltpu.make_async_copy(
                e_hbm.at[1, pl.ds(0, K)], col_v, isems[b]).wait()

            def _g(i, inner):
                for j in range(8):
                    o = i * 128 + j * 16
                    cc = col_v[pl.ds(o, 16)]
                    bond_v[pl.ds(o, 16)] = plsc.load_gather(xv, [cc])
                return inner

            lax.fori_loop(0, K // 128, _g, 0)

        @pl.when((t > 1) & (ch - 2 * NWORKERS < TOTAL_CHUNKS))
        def _():
            pltpu.make_async_copy(
                x_hbm.at[pl.ds(0, K)], bond_bufs[bn], ssems[bn]).wait()

        @pl.when(ch + NWORKERS < TOTAL_CHUNKS)
        def _():
            e0 = (ch + NWORKERS) * K
            pltpu.async_copy(
                e_hbm.at[0, pl.ds(e0, K)], row_bufs[bn], isems[bn])
            pltpu.async_copy(
                e_hbm.at[1, pl.ds(e0, K)], col_bufs[bn], isems[bn])

        @pl.when(valid)
        def _():
            pltpu.async_copy(bond_v, field_sp.at[row_v], ssems[b], add=True)

    def _triple(tp, carry):
        _phase(tp * 3, 0)
        _phase(tp * 3 + 1, 1)
        _phase(tp * 3 + 2, 2)
        return carry

    lax.fori_loop(0, MAXT_PAD // 3, _triple, 0)

    @pl.when(wid + NWORKERS * (MAXT - 1) < TOTAL_CHUNKS)
    def _():
        pltpu.make_async_copy(
            x_hbm.at[pl.ds(0, K)], bond_bufs[(MAXT - 1) % 3],
            ssems[(MAXT - 1) % 3]).wait()

    @pl.when((c == 0) & (s < 15))
    def _():
        pltpu.sync_copy(xv.at[pl.ds(s * ZCH, ZCH)],
                        out_hbm.at[2, pl.ds(s * ZCH, ZCH)])

    @pl.when((c == 0) & (s == 15))
    def _():
        pltpu.sync_copy(xv.at[pl.ds(15 * ZCH, XTAIL)],
                        out_hbm.at[2, pl.ds(15 * ZCH, XTAIL)])

    plsc.subcore_barrier()
    pltpu.sync_copy(field_sp.at[pl.ds(s * ZCH, ZCH)],
                    out_hbm.at[c, pl.ds(s * ZCH, ZCH)])


BN = 1024
GRID = -(-N // BN)


def _mlp_body(p_ref, w1_ref, b1_ref, w2_ref, b2_ref, o_ref):
    p = p_ref[...]
    xr = p[2:3, :]
    feats = jnp.concatenate(
        [xr, xr * (p[0:1, :] + p[1:2, :])], axis=0)
    h = lax.dot_general(w1_ref[...], feats, (((0,), (0,)), ((), ())),
                        preferred_element_type=jnp.float32)
    h = jnp.maximum(h + b1_ref[...], 0.0)
    o = lax.dot_general(w2_ref[...], h, (((0,), (0,)), ((), ())),
                        preferred_element_type=jnp.float32)
    o = jnp.maximum(o + b2_ref[...], 0.0)
    o_ref[...] = o.T


_mlp = pl.pallas_call(
    _mlp_body,
    grid=(GRID,),
    in_specs=[
        pl.BlockSpec((3, BN), lambda i: (0, i)),
        pl.BlockSpec((2, TD), lambda i: (0, 0)),
        pl.BlockSpec((TD, 1), lambda i: (0, 0)),
        pl.BlockSpec((TD, TD), lambda i: (0, 0)),
        pl.BlockSpec((TD, 1), lambda i: (0, 0)),
    ],
    out_specs=pl.BlockSpec((BN, TD), lambda i: (i, 0)),
    out_shape=jax.ShapeDtypeStruct((N, TD), jnp.float32),
)


def kernel(x, edge_index, W1, b1, W2, b2):
    part = _sc_field(x.reshape((N,)), edge_index.astype(jnp.int32))
    return _mlp(part, W1, b1.reshape(TD, 1), W2, b2.reshape(TD, 1))


# speedup vs baseline: 1.3829x; 1.1945x over previous
import functools

import jax
import jax.numpy as jnp
from jax import lax
from jax.experimental import pallas as pl
from jax.experimental.pallas import tpu as pltpu
from jax.experimental.pallas import tpu_sc as plsc

N = 100000
E = 6400000
TD = 16

NWORKERS = 32
ZCH = 6272
NF = 16 * ZCH
K = 2048
TOTAL_CHUNKS = E // K
MAXT = -(-TOTAL_CHUNKS // NWORKERS)
MAXT_PAD = 99
XTAIL = N - 15 * ZCH

_mesh = plsc.VectorSubcoreMesh(core_axis_name="c", subcore_axis_name="s")


@functools.partial(
    pl.kernel,
    out_type=jax.ShapeDtypeStruct((3, NF), jnp.float32),
    mesh=_mesh,
    compiler_params=pltpu.CompilerParams(
        needs_layout_passes=False,
        use_tc_tiling_on_sc=False,
    ),
    scratch_types=[
        pltpu.VMEM((N,), jnp.float32),
        pltpu.VMEM((K,), jnp.int32),
        pltpu.VMEM((K,), jnp.int32),
        pltpu.VMEM((K,), jnp.int32),
        pltpu.VMEM((K,), jnp.int32),
        pltpu.VMEM((K,), jnp.int32),
        pltpu.VMEM((K,), jnp.int32),
        pltpu.VMEM((K,), jnp.float32),
        pltpu.VMEM((K,), jnp.float32),
        pltpu.VMEM((K,), jnp.float32),
        pltpu.VMEM((ZCH,), jnp.float32),
        pltpu.VMEM_SHARED((NF,), jnp.float32),
        pltpu.SemaphoreType.DMA,
        pltpu.SemaphoreType.DMA,
        pltpu.SemaphoreType.DMA,
        pltpu.SemaphoreType.DMA,
        pltpu.SemaphoreType.DMA,
        pltpu.SemaphoreType.DMA,
        pltpu.SemaphoreType.DMA,
    ],
)
def _sc_field(x_hbm, e_hbm, out_hbm, xv, rowv0, rowv1, rowv2,
              colv0, colv1, colv2, bond0, bond1, bond2, zero_v, field_sp,
              sem_x, sem_i0, sem_i1, sem_i2, sem_s0, sem_s1, sem_s2):
    c = lax.axis_index("c")
    s = lax.axis_index("s")
    wid = s * 2 + c

    row_bufs = (rowv0, rowv1, rowv2)
    col_bufs = (colv0, colv1, colv2)
    bond_bufs = (bond0, bond1, bond2)
    isems = (sem_i0, sem_i1, sem_i2)
    ssems = (sem_s0, sem_s1, sem_s2)

    cp_x = pltpu.async_copy(x_hbm, xv, sem_x)
    pltpu.async_copy(e_hbm.at[0, pl.ds(wid * K, K)], rowv0, sem_i0)
    pltpu.async_copy(e_hbm.at[1, pl.ds(wid * K, K)], colv0, sem_i0)

    zeros16 = jnp.zeros((16,), jnp.float32)

    def _zbody(i, carry):
        zero_v[pl.ds(i * 16, 16)] = zeros16
        return carry

    lax.fori_loop(0, ZCH // 16, _zbody, 0)
    pltpu.sync_copy(zero_v, field_sp.at[pl.ds(s * ZCH, ZCH)])
    cp_x.wait()
    plsc.subcore_barrier()

    def _phase(t, b):
        ch = wid + NWORKERS * t
        valid = ch < TOTAL_CHUNKS
        bn = (b + 1) % 3
        row_v = row_bufs[b]
        col_v = col_bufs[b]
        bond_v = bond_bufs[b]

        @pl.when(valid)
        def _():
            pltpu.make_async_copy(
                e_hbm.at[0, pl.ds(0, K)], row_v, isems[b]).wait()
            pltpu.make_async_copy(
                e_hbm.at[1, pl.ds(0, K)], col_v, isems[b]).wait()

            def _g(i, inner):
                for j in range(8):
                    o = i * 128 + j * 16
                    cc = col_v[pl.ds(o, 16)]
                    bond_v[pl.ds(o, 16)] = plsc.load_gather(xv, [cc])
                return inner

            lax.fori_loop(0, K // 128, _g, 0)

        @pl.when((t > 1) & (ch - 2 * NWORKERS < TOTAL_CHUNKS))
        def _():
            pltpu.make_async_copy(
                x_hbm.at[pl.ds(0, K)], bond_bufs[bn], ssems[bn]).wait()

        @pl.when(ch + NWORKERS < TOTAL_CHUNKS)
        def _():
            e0 = (ch + NWORKERS) * K
            pltpu.async_copy(
                e_hbm.at[0, pl.ds(e0, K)], row_bufs[bn], isems[bn])
            pltpu.async_copy(
                e_hbm.at[1, pl.ds(e0, K)], col_bufs[bn], isems[bn])

        @pl.when(valid)
        def _():
            pltpu.async_copy(bond_v, field_sp.at[row_v], ssems[b], add=True)

    def _triple(tp, carry):
        _phase(tp * 3, 0)
        _phase(tp * 3 + 1, 1)
        _phase(tp * 3 + 2, 2)
        return carry

    lax.fori_loop(0, MAXT_PAD // 3, _triple, 0)

    @pl.when(wid + NWORKERS * (MAXT - 1) < TOTAL_CHUNKS)
    def _():
        pltpu.make_async_copy(
            x_hbm.at[pl.ds(0, K)], bond_bufs[(MAXT - 1) % 3],
            ssems[(MAXT - 1) % 3]).wait()

    @pl.when((c == 0) & (s < 15))
    def _():
        pltpu.sync_copy(xv.at[pl.ds(s * ZCH, ZCH)],
                        out_hbm.at[2, pl.ds(s * ZCH, ZCH)])

    @pl.when((c == 0) & (s == 15))
    def _():
        pltpu.sync_copy(xv.at[pl.ds(15 * ZCH, XTAIL)],
                        out_hbm.at[2, pl.ds(15 * ZCH, XTAIL)])

    plsc.subcore_barrier()
    pltpu.sync_copy(field_sp.at[pl.ds(s * ZCH, ZCH)],
                    out_hbm.at[c, pl.ds(s * ZCH, ZCH)])


BN = 8192
GRID = -(-N // BN)


def _mlp_body(p_ref, w1_ref, b1_ref, w2_ref, b2_ref, o_ref):
    p = p_ref[...]
    xr = p[2:3, :]
    feats = jnp.concatenate(
        [xr, xr * (p[0:1, :] + p[1:2, :])], axis=0)
    h = lax.dot_general(feats, w1_ref[...], (((0,), (0,)), ((), ())),
                        preferred_element_type=jnp.float32)
    h = jnp.maximum(h + b1_ref[...], 0.0)
    o = lax.dot_general(h, w2_ref[...], (((1,), (0,)), ((), ())),
                        preferred_element_type=jnp.float32)
    o_ref[...] = jnp.maximum(o + b2_ref[...], 0.0)


_mlp = pl.pallas_call(
    _mlp_body,
    grid=(GRID,),
    in_specs=[
        pl.BlockSpec((3, BN), lambda i: (0, i)),
        pl.BlockSpec((2, TD), lambda i: (0, 0)),
        pl.BlockSpec((1, TD), lambda i: (0, 0)),
        pl.BlockSpec((TD, TD), lambda i: (0, 0)),
        pl.BlockSpec((1, TD), lambda i: (0, 0)),
    ],
    out_specs=pl.BlockSpec((BN, TD), lambda i: (i, 0)),
    out_shape=jax.ShapeDtypeStruct((N, TD), jnp.float32),
)


def kernel(x, edge_index, W1, b1, W2, b2):
    part = _sc_field(x.reshape((N,)), edge_index.astype(jnp.int32))
    return _mlp(part, W1, b1.reshape(1, TD), W2, b2.reshape(1, TD))
